# baseline (speedup 1.0000x reference)
import jax
import jax.numpy as jnp
from jax import lax
from jax.experimental import pallas as pl
from jax.experimental.pallas import tpu as pltpu

_HEAD = 0
_TAIL = 1
_BSZ, _SEQ, _HID = 128, 512, 1024


def _fused_body(ids_ref, lhs_ref, w_ref, b_ref, log_ref, ent_ref,
                idx_vmem, idx_smem, sem, gsems):
    ids = ids_ref[...]
    col = lax.broadcasted_iota(jnp.int32, (_BSZ, _SEQ), 1)
    idx_vmem[0, :] = jnp.min(jnp.where(ids == _HEAD, col, _SEQ), axis=1)
    idx_vmem[1, :] = jnp.min(jnp.where(ids == _TAIL, col, _SEQ), axis=1)
    bounce = pltpu.make_async_copy(idx_vmem, idx_smem, sem)
    bounce.start()
    bounce.wait()
    copies = []
    for r in range(_BSZ):
        hp = idx_smem[0, r]
        tp = idx_smem[1, r]
        ch = pltpu.make_async_copy(
            lhs_ref.at[r, pl.ds(hp, 1), :],
            ent_ref.at[pl.ds(r, 1), pl.ds(0, _HID)], gsems.at[(2 * r) % 8])
        ct = pltpu.make_async_copy(
            lhs_ref.at[r, pl.ds(tp, 1), :],
            ent_ref.at[pl.ds(r, 1), pl.ds(_HID, _HID)], gsems.at[(2 * r + 1) % 8])
        ch.start()
        ct.start()
        copies.append(ch)
        copies.append(ct)
    for c in copies:
        c.wait()
    log_ref[...] = lax.dot_general(
        w_ref[...], ent_ref[...],
        dimension_numbers=(((1,), (1,)), ((), ())),
        preferred_element_type=jnp.float32,
    ) + jnp.transpose(b_ref[...])


def kernel(input_ids, last_hidden_state, W, b):
    nlab = W.shape[0]
    logits, entity = pl.pallas_call(
        _fused_body,
        in_specs=[
            pl.BlockSpec(memory_space=pltpu.VMEM),
            pl.BlockSpec(memory_space=pl.ANY),
            pl.BlockSpec(memory_space=pltpu.VMEM),
            pl.BlockSpec(memory_space=pltpu.VMEM),
        ],
        out_specs=[
            pl.BlockSpec(memory_space=pltpu.VMEM),
            pl.BlockSpec(memory_space=pltpu.VMEM),
        ],
        out_shape=(
            jax.ShapeDtypeStruct((nlab, _BSZ), jnp.float32),
            jax.ShapeDtypeStruct((_BSZ, 2 * _HID), jnp.float32),
        ),
        scratch_shapes=[
            pltpu.VMEM((8, _BSZ), jnp.int32),
            pltpu.SMEM((8, _BSZ), jnp.int32),
            pltpu.SemaphoreType.DMA,
            pltpu.SemaphoreType.DMA((8,)),
        ],
    )(input_ids, last_hidden_state, W, b.reshape(1, nlab))
    return (logits.T, entity)


# speedup vs baseline: 1.0288x; 1.0288x over previous
import jax
import jax.numpy as jnp
from jax import lax
from jax.experimental import pallas as pl
from jax.experimental.pallas import tpu as pltpu

_HEAD = 0
_TAIL = 1
_BSZ, _SEQ, _HID = 128, 512, 1024


def _fused_body(ids_ref, lhs_ref, w_ref, b_ref, log_ref, ent_ref,
                idx_vmem, idx_smem, sem):
    ids = ids_ref[...]
    col = lax.broadcasted_iota(jnp.int32, (_BSZ, _SEQ), 1)
    idx_vmem[0, :] = jnp.min(jnp.where(ids == _HEAD, col, _SEQ), axis=1)
    idx_vmem[1, :] = jnp.min(jnp.where(ids == _TAIL, col, _SEQ), axis=1)
    bounce = pltpu.make_async_copy(idx_vmem, idx_smem, sem)
    bounce.start()
    bounce.wait()
    for r in range(_BSZ):
        hp = idx_smem[0, r]
        tp = idx_smem[1, r]
        pltpu.make_async_copy(
            lhs_ref.at[r, pl.ds(hp, 1), :],
            ent_ref.at[pl.ds(r, 1), pl.ds(0, _HID)], sem).start()
        pltpu.make_async_copy(
            lhs_ref.at[r, pl.ds(tp, 1), :],
            ent_ref.at[pl.ds(r, 1), pl.ds(_HID, _HID)], sem).start()
    pltpu.make_async_copy(
        lhs_ref.at[0, pl.ds(0, _BSZ), :],
        ent_ref.at[pl.ds(0, _BSZ), pl.ds(0, _HID)], sem).wait()
    pltpu.make_async_copy(
        lhs_ref.at[0, pl.ds(0, _BSZ), :],
        ent_ref.at[pl.ds(0, _BSZ), pl.ds(_HID, _HID)], sem).wait()
    log_ref[...] = lax.dot_general(
        w_ref[...], ent_ref[...],
        dimension_numbers=(((1,), (1,)), ((), ())),
        preferred_element_type=jnp.float32,
    ) + jnp.transpose(b_ref[...])


def kernel(input_ids, last_hidden_state, W, b):
    nlab = W.shape[0]
    logits, entity = pl.pallas_call(
        _fused_body,
        in_specs=[
            pl.BlockSpec(memory_space=pltpu.VMEM),
            pl.BlockSpec(memory_space=pl.ANY),
            pl.BlockSpec(memory_space=pltpu.VMEM),
            pl.BlockSpec(memory_space=pltpu.VMEM),
        ],
        out_specs=[
            pl.BlockSpec(memory_space=pltpu.VMEM),
            pl.BlockSpec(memory_space=pltpu.VMEM),
        ],
        out_shape=(
            jax.ShapeDtypeStruct((nlab, _BSZ), jnp.float32),
            jax.ShapeDtypeStruct((_BSZ, 2 * _HID), jnp.float32),
        ),
        scratch_shapes=[
            pltpu.VMEM((8, _BSZ), jnp.int32),
            pltpu.SMEM((8, _BSZ), jnp.int32),
            pltpu.SemaphoreType.DMA,
        ],
    )(input_ids, last_hidden_state, W, b.reshape(1, nlab))
    return (logits.T, entity)


# speedup vs baseline: 1.0459x; 1.0166x over previous
import jax
import jax.numpy as jnp
from jax import lax
from jax.experimental import pallas as pl
from jax.experimental.pallas import tpu as pltpu

_HEAD = 0
_TAIL = 1
_BSZ, _SEQ, _HID = 128, 512, 1024


def _fused_body(ids_ref, lhs_ref, w_ref, b_ref, log_ref, ent_ref,
                idx_vmem, idx_smem, w_vmem, sem, wsem, bsem):
    w_load = pltpu.make_async_copy(w_ref, w_vmem, wsem)
    w_load.start()
    ids = ids_ref[...]
    col = lax.broadcasted_iota(jnp.int32, (_BSZ, _SEQ), 1)
    idx_vmem[0, :] = jnp.min(jnp.where(ids == _HEAD, col, _SEQ), axis=1)
    bh = pltpu.make_async_copy(
        idx_vmem.at[pl.ds(0, 1), :], idx_smem.at[pl.ds(0, 1), :], bsem)
    bh.start()
    idx_vmem[1, :] = jnp.min(jnp.where(ids == _TAIL, col, _SEQ), axis=1)
    bt = pltpu.make_async_copy(
        idx_vmem.at[pl.ds(1, 1), :], idx_smem.at[pl.ds(1, 1), :], bsem)
    bt.start()
    bh.wait()
    for r in range(_BSZ):
        hp = idx_smem[0, r]
        pltpu.make_async_copy(
            lhs_ref.at[r, pl.ds(hp, 1), :],
            ent_ref.at[pl.ds(r, 1), pl.ds(0, _HID)], sem).start()
    bt.wait()
    for r in range(_BSZ):
        tp = idx_smem[1, r]
        pltpu.make_async_copy(
            lhs_ref.at[r, pl.ds(tp, 1), :],
            ent_ref.at[pl.ds(r, 1), pl.ds(_HID, _HID)], sem).start()
    pltpu.make_async_copy(
        lhs_ref.at[0, pl.ds(0, _BSZ), :],
        ent_ref.at[pl.ds(0, _BSZ), pl.ds(0, _HID)], sem).wait()
    pltpu.make_async_copy(
        lhs_ref.at[0, pl.ds(0, _BSZ), :],
        ent_ref.at[pl.ds(0, _BSZ), pl.ds(_HID, _HID)], sem).wait()
    w_load.wait()
    log_ref[...] = lax.dot_general(
        w_vmem[...], ent_ref[...],
        dimension_numbers=(((1,), (1,)), ((), ())),
        preferred_element_type=jnp.float32,
    ) + jnp.transpose(b_ref[...])


def kernel(input_ids, last_hidden_state, W, b):
    nlab = W.shape[0]
    logits, entity = pl.pallas_call(
        _fused_body,
        in_specs=[
            pl.BlockSpec(memory_space=pltpu.VMEM),
            pl.BlockSpec(memory_space=pl.ANY),
            pl.BlockSpec(memory_space=pl.ANY),
            pl.BlockSpec(memory_space=pltpu.VMEM),
        ],
        out_specs=[
            pl.BlockSpec(memory_space=pltpu.VMEM),
            pl.BlockSpec(memory_space=pltpu.VMEM),
        ],
        out_shape=(
            jax.ShapeDtypeStruct((nlab, _BSZ), jnp.float32),
            jax.ShapeDtypeStruct((_BSZ, 2 * _HID), jnp.float32),
        ),
        scratch_shapes=[
            pltpu.VMEM((8, _BSZ), jnp.int32),
            pltpu.SMEM((8, _BSZ), jnp.int32),
            pltpu.VMEM((23, 2 * _HID), jnp.float32),
            pltpu.SemaphoreType.DMA,
            pltpu.SemaphoreType.DMA,
            pltpu.SemaphoreType.DMA,
        ],
    )(input_ids, last_hidden_state, W, b.reshape(1, nlab))
    return (logits.T, entity)
